# speedup vs baseline: 3.0896x; 3.0896x over previous
import jax
import jax.numpy as jnp
from jax import lax
from jax.experimental import pallas as pl
from jax.experimental.pallas import tpu as pltpu

_VMEM_LIMIT = 64 * 1024 * 1024


def _proj_kernel(x_ref, w_ref, qkv_ref):
    qkv_ref[...] = jnp.dot(
        x_ref[...], w_ref[...], preferred_element_type=jnp.float32
    ).astype(qkv_ref.dtype)


def _attn_kernel(q_ref, k_ref, v_ref, o_ref):
    s = lax.dot_general(
        q_ref[...], k_ref[...], (((1,), (1,)), ((), ())),
        preferred_element_type=jnp.float32)
    m = jnp.max(s, axis=-1, keepdims=True)
    p = jnp.exp(s - m)
    l = jnp.sum(p, axis=-1, keepdims=True)
    o = jnp.dot(p, v_ref[...], preferred_element_type=jnp.float32)
    o_ref[...] = (o * pl.reciprocal(l, approx=False)).astype(o_ref.dtype)


def kernel(x, w_qkv):
    seq, d_in = x.shape
    d = w_qkv.shape[1] // 3
    out_dtype = x.dtype

    tp = 1024
    qkv = pl.pallas_call(
        _proj_kernel,
        out_shape=jax.ShapeDtypeStruct((seq, 3 * d), jnp.float32),
        grid=(seq // tp,),
        in_specs=[
            pl.BlockSpec((tp, d_in), lambda i: (i, 0)),
            pl.BlockSpec((d_in, 3 * d), lambda i: (0, 0)),
        ],
        out_specs=pl.BlockSpec((tp, 3 * d), lambda i: (i, 0)),
        compiler_params=pltpu.CompilerParams(
            dimension_semantics=("parallel",),
            vmem_limit_bytes=_VMEM_LIMIT),
    )(x, w_qkv)

    tq = 256
    out = pl.pallas_call(
        _attn_kernel,
        out_shape=jax.ShapeDtypeStruct((seq, d), out_dtype),
        grid=(seq // tq,),
        in_specs=[
            pl.BlockSpec((tq, d), lambda i: (i, 0)),
            pl.BlockSpec((seq, d), lambda i: (0, 1)),
            pl.BlockSpec((seq, d), lambda i: (0, 2)),
        ],
        out_specs=pl.BlockSpec((tq, d), lambda i: (i, 0)),
        compiler_params=pltpu.CompilerParams(
            dimension_semantics=("parallel",),
            vmem_limit_bytes=_VMEM_LIMIT),
    )(qkv, qkv, qkv)

    return out


# speedup vs baseline: 4.4836x; 1.4512x over previous
import jax
import jax.numpy as jnp
from jax import lax
from jax.experimental import pallas as pl
from jax.experimental.pallas import tpu as pltpu

_VMEM_LIMIT = 64 * 1024 * 1024


def _proj_kernel(x_ref, w_ref, qkv_ref):
    qkv_ref[...] = jnp.dot(
        x_ref[...], w_ref[...], preferred_element_type=jnp.float32
    ).astype(qkv_ref.dtype)


def _attn_kernel(q_ref, k_ref, v_ref, o_ref):
    s = lax.dot_general(
        q_ref[...].astype(jnp.bfloat16), k_ref[...].astype(jnp.bfloat16),
        (((1,), (1,)), ((), ())),
        preferred_element_type=jnp.float32)
    m = jnp.max(s, axis=-1, keepdims=True)
    p = jnp.exp(s - m)
    l = jnp.sum(p, axis=-1, keepdims=True)
    o = jnp.dot(p.astype(jnp.bfloat16), v_ref[...].astype(jnp.bfloat16),
                preferred_element_type=jnp.float32)
    o_ref[...] = (o * pl.reciprocal(l, approx=False)).astype(o_ref.dtype)


def kernel(x, w_qkv):
    seq, d_in = x.shape
    d = w_qkv.shape[1] // 3
    out_dtype = x.dtype

    tp = 1024
    qkv = pl.pallas_call(
        _proj_kernel,
        out_shape=jax.ShapeDtypeStruct((seq, 3 * d), jnp.float32),
        grid=(seq // tp,),
        in_specs=[
            pl.BlockSpec((tp, d_in), lambda i: (i, 0)),
            pl.BlockSpec((d_in, 3 * d), lambda i: (0, 0)),
        ],
        out_specs=pl.BlockSpec((tp, 3 * d), lambda i: (i, 0)),
        compiler_params=pltpu.CompilerParams(
            dimension_semantics=("parallel",),
            vmem_limit_bytes=_VMEM_LIMIT),
    )(x, w_qkv)

    tq = 256
    out = pl.pallas_call(
        _attn_kernel,
        out_shape=jax.ShapeDtypeStruct((seq, d), out_dtype),
        grid=(seq // tq,),
        in_specs=[
            pl.BlockSpec((tq, d), lambda i: (i, 0)),
            pl.BlockSpec((seq, d), lambda i: (0, 1)),
            pl.BlockSpec((seq, d), lambda i: (0, 2)),
        ],
        out_specs=pl.BlockSpec((tq, d), lambda i: (i, 0)),
        compiler_params=pltpu.CompilerParams(
            dimension_semantics=("parallel",),
            vmem_limit_bytes=_VMEM_LIMIT),
    )(qkv, qkv, qkv)

    return out
